# baseline (speedup 1.0000x reference)
import functools

import jax
import jax.numpy as jnp
from jax import lax
from jax.experimental import pallas as pl
from jax.experimental.pallas import tpu as pltpu
from jax.experimental.pallas import tpu_sc as plsc

N = 10000
E = 320000
D = 128

NT = 16
NW = 32
C = 64
NCH = 160
NPAIR = NCH // 2
EPAD = NW * NCH * C
NP = 10240
RPT = NP // NT
EPW = EPAD // NW
PAD = EPAD - E
RP_LAST = (E - (NW - 1) * EPW) // (2 * C)

_mesh = plsc.VectorSubcoreMesh(core_axis_name="c", subcore_axis_name="s")


@functools.partial(
    pl.kernel,
    mesh=_mesh,
    out_type=jax.ShapeDtypeStruct((2 * N, D), jnp.float32),
    scratch_types=[
        pltpu.VMEM((4, 2 * C), jnp.int32),
        pltpu.VMEM((4, 2 * C), jnp.int32),
        pltpu.VMEM((C, D), jnp.float32),
        pltpu.VMEM((C, D), jnp.float32),
        pltpu.VMEM((C, D), jnp.float32),
        pltpu.VMEM((C, D), jnp.float32),
        pltpu.VMEM_SHARED((NP, D), jnp.float32),
        pltpu.SemaphoreType.DMA,
        pltpu.SemaphoreType.DMA,
        pltpu.SemaphoreType.DMA,
        pltpu.SemaphoreType.DMA,
        pltpu.SemaphoreType.DMA,
        pltpu.SemaphoreType.DMA,
    ],
)
def _sc_segment_sum(x_hbm, src_hbm, dst_hbm, psrc_hbm, pdst_hbm,
                    zero_hbm, out_hbm,
                    sidx, didx, rows0, rows1, rows2, rows3, agg,
                    sem_g0, sem_g1, sem_s0, sem_s1, sem_si, sem_di):
    c = lax.axis_index("c")
    s = lax.axis_index("s")
    w = c * NT + s

    def gather(p, h, buf, sem):
        return pltpu.async_copy(
            x_hbm.at[sidx.at[p % 4, pl.ds(h * C, C)]], buf, sem)

    def scatter(buf, p, h, sem):
        return pltpu.async_copy(
            buf, agg.at[didx.at[p % 4, pl.ds(h * C, C)]], sem, add=True)

    def pf(p, slot):
        is_main = jnp.logical_or(w < NW - 1, p < RP_LAST)

        @pl.when(is_main)
        def _():
            pltpu.async_copy(src_hbm.at[pl.ds(w * EPW + p * 2 * C, 2 * C)],
                             sidx.at[slot], sem_si)
            pltpu.async_copy(dst_hbm.at[pl.ds(w * EPW + p * 2 * C, 2 * C)],
                             didx.at[slot], sem_di)

        @pl.when(jnp.logical_not(is_main))
        def _():
            pltpu.async_copy(psrc_hbm.at[pl.ds((p - RP_LAST) * 2 * C, 2 * C)],
                             sidx.at[slot], sem_si)
            pltpu.async_copy(pdst_hbm.at[pl.ds((p - RP_LAST) * 2 * C, 2 * C)],
                             didx.at[slot], sem_di)

    def drain_g(sem, buf):
        pltpu.make_async_copy(zero_hbm.at[pl.ds(0, C)], buf, sem).wait()

    def drain_s(sem, buf):
        pltpu.make_async_copy(zero_hbm.at[pl.ds(0, C)], buf, sem).wait()

    def drain_pf():
        pltpu.make_async_copy(src_hbm.at[pl.ds(0, 2 * C)], sidx.at[0],
                              sem_si).wait()
        pltpu.make_async_copy(dst_hbm.at[pl.ds(0, 2 * C)], didx.at[0],
                              sem_di).wait()

    cp_z = pltpu.async_copy(zero_hbm,
                            agg.at[pl.ds(s * RPT, RPT)], sem_s0)

    pltpu.sync_copy(src_hbm.at[pl.ds(w * EPW, 2 * C)], sidx.at[0])
    pltpu.sync_copy(dst_hbm.at[pl.ds(w * EPW, 2 * C)], didx.at[0])
    pltpu.async_copy(src_hbm.at[pl.ds(w * EPW + 2 * C, 2 * C)],
                     sidx.at[1], sem_si)
    pltpu.async_copy(dst_hbm.at[pl.ds(w * EPW + 2 * C, 2 * C)],
                     didx.at[1], sem_di)

    cp_z.wait()
    plsc.subcore_barrier()

    g_a = gather(0, 0, rows0, sem_g0)
    g_b = gather(0, 1, rows1, sem_g1)

    g_a.wait()
    scatter(rows0, 0, 0, sem_s0)
    drain_pf()
    gather(1, 0, rows2, sem_g0)
    g_b.wait()
    scatter(rows1, 0, 1, sem_s1)
    gather(1, 1, rows3, sem_g1)
    pf(2, 2)

    def body(q, carry):
        p = 2 * q + 1
        drain_g(sem_g0, rows2)
        drain_s(sem_s0, rows0)
        s_a = scatter(rows2, p, 0, sem_s0)
        drain_pf()
        g_a = gather(p + 1, 0, rows0, sem_g0)
        drain_g(sem_g1, rows3)
        drain_s(sem_s1, rows1)
        s_b = scatter(rows3, p, 1, sem_s1)
        g_b = gather(p + 1, 1, rows1, sem_g1)
        pf(lax.min(p + 2, NPAIR - 1), (p + 2) % 4)
        g_a.wait()
        s_a.wait()
        scatter(rows0, p + 1, 0, sem_s0)
        drain_pf()
        gather(lax.min(p + 2, NPAIR - 1), 0, rows2, sem_g0)
        g_b.wait()
        s_b.wait()
        scatter(rows1, p + 1, 1, sem_s1)
        gather(lax.min(p + 2, NPAIR - 1), 1, rows3, sem_g1)
        pf(lax.min(p + 3, NPAIR - 1), (p + 3) % 4)
        return carry

    lax.fori_loop(0, (NPAIR - 2) // 2, body, 0)

    drain_g(sem_g0, rows2)
    drain_s(sem_s0, rows0)
    s_a = scatter(rows2, NPAIR - 1, 0, sem_s0)
    drain_g(sem_g1, rows3)
    drain_s(sem_s1, rows1)
    s_b = scatter(rows3, NPAIR - 1, 1, sem_s1)
    drain_pf()
    s_a.wait()
    s_b.wait()

    plsc.subcore_barrier()

    off = lax.min(s * RPT, N - RPT)
    pltpu.sync_copy(agg.at[pl.ds(off, RPT)],
                    out_hbm.at[pl.ds(c * N + off, RPT)])


def _mlp_block(h, W, b, g, be):
    h = lax.dot_general(h, W, (((1,), (1,)), ((), ())),
                        preferred_element_type=jnp.float32) + b
    m = jnp.mean(h, axis=0, keepdims=True)
    v = jnp.mean((h - m) ** 2, axis=0, keepdims=True)
    h = (h - m) / jnp.sqrt(v + 1e-5) * g + be
    return h * jax.nn.sigmoid(h)


def _tc_mlp_body(parts, x, eps, W1, b1, g1, be1, W2, b2, g2, be2, o):
    h = (parts[pl.ds(0, N), :] + parts[pl.ds(N, N), :]
         + (1.0 + eps[0, 0]) * x[...])
    h = _mlp_block(h, W1[...], b1[...], g1[...], be1[...])
    h = _mlp_block(h, W2[...], b2[...], g2[...], be2[...])
    o[...] = h


def kernel(x, edge_index, eps, W1, b1, g1, be1, W2, b2, g2, be2):
    src = edge_index[0].astype(jnp.int32)
    dst = edge_index[1].astype(jnp.int32)
    iot = jnp.arange(PAD, dtype=jnp.int32)
    psrc = iot % N
    pdst = N + iot % (NP - N)
    zeros = jnp.zeros((RPT, D), jnp.float32)

    partials = _sc_segment_sum(x, src, dst, psrc, pdst, zeros)

    out = pl.pallas_call(
        _tc_mlp_body,
        out_shape=jax.ShapeDtypeStruct((N, D), jnp.float32),
    )(partials, x, eps.reshape(1, 1),
      W1, b1.reshape(1, D), g1.reshape(1, D), be1.reshape(1, D),
      W2, b2.reshape(1, D), g2.reshape(1, D), be2.reshape(1, D))
    return out
